# baseline (device time: 13106 ns/iter reference)
import jax
import jax.numpy as jnp
from jax import lax
from jax.experimental import pallas as pl
from jax.experimental.pallas import tpu as pltpu

N_DEV = 4
B, SQ, SKV, DH = 2, 128, 128, 64
H_PER = 4
D_MODEL = 512
CHUNK = D_MODEL // N_DEV
ROWS = B * SQ


def kernel(x, Wq, K_ext, V_ext, Wo):
    X2 = x.reshape(ROWS, D_MODEL)
    X2 = pltpu.with_memory_space_constraint(X2, pltpu.MemorySpace.HBM)
    me_out = lax.axis_index("i")
    K2 = lax.dynamic_slice_in_dim(K_ext, me_out * H_PER, H_PER, axis=2)
    K2 = K2.reshape(B, SKV, H_PER * DH).astype(jnp.bfloat16)
    V2 = lax.dynamic_slice_in_dim(V_ext, me_out * H_PER, H_PER, axis=2)
    V2 = V2.reshape(B, SKV, H_PER * DH).astype(jnp.bfloat16)

    def body(x_ref, wq_ref, k_ref, v_ref, wo_ref, out_ref,
             x_vmem,
             rs_send, rs_recv, ag_send, ag_recv,
             x_sem,
             rs_send_sems, rs_recv_sems, ag_send_sems, ag_recv_sems):
        me = lax.axis_index("i")

        x_dmas = []
        for b in range(B):
            dma = pltpu.make_async_copy(
                x_ref.at[pl.ds(b * SQ, SQ), :],
                x_vmem.at[pl.ds(b * SQ, SQ), :], x_sem.at[b])
            dma.start()
            x_dmas.append(dma)

        barrier_sem = pltpu.get_barrier_semaphore()
        for rel in range(1, N_DEV):
            peer = lax.rem(me + rel, N_DEV)
            pl.semaphore_signal(
                barrier_sem, inc=1,
                device_id=(peer,), device_id_type=pl.DeviceIdType.MESH,
            )

        def qproj(b):
            x_dmas[b].wait()
            return jnp.dot(x_vmem[pl.ds(b * SQ, SQ), :], wq_ref[...],
                           preferred_element_type=jnp.float32)

        q_parts = [None, None]
        q_parts[0] = qproj(0)

        def attn(b):
            parts = []
            for h in range(H_PER):
                qh = q_parts[b][:, h * DH:(h + 1) * DH]
                kh = k_ref[b, :, h * DH:(h + 1) * DH]
                vh = v_ref[b, :, h * DH:(h + 1) * DH]
                s = lax.dot_general(
                    qh.astype(jnp.bfloat16), kh, (((1,), (1,)), ((), ())),
                    preferred_element_type=jnp.float32) * 0.125
                w = jnp.exp(s)
                inv = 1.0 / jnp.sum(w, axis=1, keepdims=True)
                ctx_h = jnp.dot(w.astype(jnp.bfloat16), vh,
                                preferred_element_type=jnp.float32)
                parts.append(ctx_h * inv)
            return jnp.concatenate(parts, axis=1)

        dests = [lax.rem(me + rel, N_DEV) for rel in range(1, N_DEV)]
        wo_cols = [wo_ref[:, pl.ds(d * CHUNK, CHUNK)] for d in dests]
        wo_own = wo_ref[:, pl.ds(me * CHUNK, CHUNK)]

        ctx0 = attn(0)
        pl.semaphore_wait(barrier_sem, N_DEV - 1)

        rs_descs = [[], []]
        ag_descs = [[], []]
        own = [None, None]

        def rs_round(r, ctx_r):
            for j in (1, 0, 2):
                d = dests[j]
                rs_send[r, j] = jnp.dot(
                    ctx_r, wo_cols[j],
                    preferred_element_type=jnp.float32).astype(jnp.bfloat16)
                rd = pltpu.make_async_remote_copy(
                    src_ref=rs_send.at[r, j],
                    dst_ref=rs_recv.at[r, j],
                    send_sem=rs_send_sems.at[r, j],
                    recv_sem=rs_recv_sems.at[r, j],
                    device_id=(d,),
                    device_id_type=pl.DeviceIdType.MESH,
                )
                rd.start()
                rs_descs[r].append((j, rd))
            own[r] = jnp.dot(ctx_r, wo_own, preferred_element_type=jnp.float32)

        def reduce_and_ag(r):
            for _, rd in rs_descs[r]:
                rd.wait_recv()
            red = (own[r]
                   + rs_recv[r, 0].astype(jnp.float32)
                   + rs_recv[r, 1].astype(jnp.float32)
                   + rs_recv[r, 2].astype(jnp.float32))
            ag_send[r] = red.astype(jnp.bfloat16)
            out_ref[r * SQ:(r + 1) * SQ, pl.ds(me * CHUNK, CHUNK)] = red
            for j in (1, 0, 2):
                d = dests[j]
                rd = pltpu.make_async_remote_copy(
                    src_ref=ag_send.at[r],
                    dst_ref=ag_recv.at[r, j],
                    send_sem=ag_send_sems.at[r, j],
                    recv_sem=ag_recv_sems.at[r, j],
                    device_id=(d,),
                    device_id_type=pl.DeviceIdType.MESH,
                )
                rd.start()
                ag_descs[r].append((j, rd))

        rs_round(0, ctx0)
        q_parts[1] = qproj(1)
        ctx1 = attn(1)
        rs_round(1, ctx1)
        reduce_and_ag(0)
        reduce_and_ag(1)

        for r in range(B):
            for j, rd in ag_descs[r]:
                rd.wait_recv()
                src = lax.rem(me + N_DEV - 1 - j, N_DEV)
                out_ref[r * SQ:(r + 1) * SQ, pl.ds(src * CHUNK, CHUNK)] = \
                    ag_recv[r, j].astype(jnp.float32)

        for r in range(B):
            for _, rd in rs_descs[r] + ag_descs[r]:
                rd.wait_send()

    out2 = pl.pallas_call(
        body,
        out_shape=jax.ShapeDtypeStruct((ROWS, D_MODEL), jnp.float32),
        in_specs=[
            pl.BlockSpec(memory_space=pltpu.MemorySpace.HBM),
            pl.BlockSpec(memory_space=pltpu.VMEM),
            pl.BlockSpec(memory_space=pltpu.VMEM),
            pl.BlockSpec(memory_space=pltpu.VMEM),
            pl.BlockSpec(memory_space=pltpu.VMEM),
        ],
        out_specs=pl.BlockSpec(memory_space=pltpu.VMEM),
        scratch_shapes=[
            pltpu.VMEM((ROWS, D_MODEL), jnp.float32),
            pltpu.VMEM((B, N_DEV - 1, SQ, CHUNK), jnp.bfloat16),
            pltpu.VMEM((B, N_DEV - 1, SQ, CHUNK), jnp.bfloat16),
            pltpu.VMEM((B, SQ, CHUNK), jnp.bfloat16),
            pltpu.VMEM((B, N_DEV - 1, SQ, CHUNK), jnp.bfloat16),
            pltpu.SemaphoreType.DMA((B,)),
            pltpu.SemaphoreType.DMA((B, N_DEV - 1)),
            pltpu.SemaphoreType.DMA((B, N_DEV - 1)),
            pltpu.SemaphoreType.DMA((B, N_DEV - 1)),
            pltpu.SemaphoreType.DMA((B, N_DEV - 1)),
        ],
        compiler_params=pltpu.CompilerParams(collective_id=0),
    )(X2, Wq, K2, V2, Wo)
    return out2.reshape(B, SQ, D_MODEL)


# device time: 12872 ns/iter; 1.0182x vs baseline; 1.0182x over previous
import jax
import jax.numpy as jnp
from jax import lax
from jax.experimental import pallas as pl
from jax.experimental.pallas import tpu as pltpu

N_DEV = 4
B, SQ, SKV, DH = 2, 128, 128, 64
H_PER = 4
D_MODEL = 512
CHUNK = D_MODEL // N_DEV
ROWS = B * SQ


def kernel(x, Wq, K_ext, V_ext, Wo):
    X2 = x.reshape(ROWS, D_MODEL)
    X2 = pltpu.with_memory_space_constraint(X2, pltpu.MemorySpace.HBM)
    me_out = lax.axis_index("i")
    K2 = lax.dynamic_slice_in_dim(K_ext, me_out * H_PER, H_PER, axis=2)
    K2 = K2.reshape(B, SKV, H_PER * DH).astype(jnp.bfloat16)
    V2 = lax.dynamic_slice_in_dim(V_ext, me_out * H_PER, H_PER, axis=2)
    V2 = V2.reshape(B, SKV, H_PER * DH).astype(jnp.bfloat16)

    def body(x_ref, wq_ref, k_ref, v_ref, wo_ref, out_ref,
             x_vmem,
             rs_send, rs_recv, ag_send, ag_recv,
             x_sem,
             rs_send_sems, rs_recv_sems, ag_send_sems, ag_recv_sems):
        me = lax.axis_index("i")

        x_dma = pltpu.make_async_copy(x_ref, x_vmem, x_sem)
        x_dma.start()

        barrier_sem = pltpu.get_barrier_semaphore()
        for rel in range(1, N_DEV):
            peer = lax.rem(me + rel, N_DEV)
            pl.semaphore_signal(
                barrier_sem, inc=1,
                device_id=(peer,), device_id_type=pl.DeviceIdType.MESH,
            )

        x_dma.wait()
        q = jnp.dot(x_vmem[...], wq_ref[...],
                    preferred_element_type=jnp.float32)

        def attn(b):
            parts = []
            for h in range(H_PER):
                qh = q[b * SQ:(b + 1) * SQ, h * DH:(h + 1) * DH]
                kh = k_ref[b, :, h * DH:(h + 1) * DH]
                vh = v_ref[b, :, h * DH:(h + 1) * DH]
                s = lax.dot_general(
                    qh.astype(jnp.bfloat16), kh, (((1,), (1,)), ((), ())),
                    preferred_element_type=jnp.float32) * 0.125
                w = jnp.exp(s)
                inv = 1.0 / jnp.sum(w, axis=1, keepdims=True)
                ctx_h = jnp.dot(w.astype(jnp.bfloat16), vh,
                                preferred_element_type=jnp.float32)
                parts.append(ctx_h * inv)
            return jnp.concatenate(parts, axis=1)

        dests = [lax.rem(me + rel, N_DEV) for rel in range(1, N_DEV)]
        wo_cols = [wo_ref[:, pl.ds(d * CHUNK, CHUNK)] for d in dests]
        wo_own = wo_ref[:, pl.ds(me * CHUNK, CHUNK)]

        ctx0 = attn(0)
        pl.semaphore_wait(barrier_sem, N_DEV - 1)

        rs_descs = [[], []]
        ag_descs = [[], []]
        own = [None, None]

        def rs_round(r, ctx_r):
            for j in (1, 0, 2):
                d = dests[j]
                rs_send[r, j] = jnp.dot(
                    ctx_r, wo_cols[j],
                    preferred_element_type=jnp.float32).astype(jnp.bfloat16)
                rd = pltpu.make_async_remote_copy(
                    src_ref=rs_send.at[r, j],
                    dst_ref=rs_recv.at[r, j],
                    send_sem=rs_send_sems.at[r, j],
                    recv_sem=rs_recv_sems.at[r, j],
                    device_id=(d,),
                    device_id_type=pl.DeviceIdType.MESH,
                )
                rd.start()
                rs_descs[r].append((j, rd))
            own[r] = jnp.dot(ctx_r, wo_own, preferred_element_type=jnp.float32)

        def reduce_and_ag(r):
            for _, rd in rs_descs[r]:
                rd.wait_recv()
            red = (own[r]
                   + rs_recv[r, 0].astype(jnp.float32)
                   + rs_recv[r, 1].astype(jnp.float32)
                   + rs_recv[r, 2].astype(jnp.float32))
            ag_send[r] = red.astype(jnp.bfloat16)
            out_ref[r * SQ:(r + 1) * SQ, pl.ds(me * CHUNK, CHUNK)] = red
            for j in (1, 0, 2):
                d = dests[j]
                rd = pltpu.make_async_remote_copy(
                    src_ref=ag_send.at[r],
                    dst_ref=ag_recv.at[r, j],
                    send_sem=ag_send_sems.at[r, j],
                    recv_sem=ag_recv_sems.at[r, j],
                    device_id=(d,),
                    device_id_type=pl.DeviceIdType.MESH,
                )
                rd.start()
                ag_descs[r].append((j, rd))

        rs_round(0, ctx0)
        ctx1 = attn(1)
        rs_round(1, ctx1)
        reduce_and_ag(0)
        reduce_and_ag(1)

        for r in range(B):
            for j, rd in ag_descs[r]:
                rd.wait_recv()
                src = lax.rem(me + N_DEV - 1 - j, N_DEV)
                out_ref[r * SQ:(r + 1) * SQ, pl.ds(src * CHUNK, CHUNK)] = \
                    ag_recv[r, j].astype(jnp.float32)

        for r in range(B):
            for _, rd in rs_descs[r] + ag_descs[r]:
                rd.wait_send()

    out2 = pl.pallas_call(
        body,
        out_shape=jax.ShapeDtypeStruct((ROWS, D_MODEL), jnp.float32),
        in_specs=[
            pl.BlockSpec(memory_space=pltpu.MemorySpace.HBM),
            pl.BlockSpec(memory_space=pltpu.VMEM),
            pl.BlockSpec(memory_space=pltpu.VMEM),
            pl.BlockSpec(memory_space=pltpu.VMEM),
            pl.BlockSpec(memory_space=pltpu.VMEM),
        ],
        out_specs=pl.BlockSpec(memory_space=pltpu.VMEM),
        scratch_shapes=[
            pltpu.VMEM((ROWS, D_MODEL), jnp.float32),
            pltpu.VMEM((B, N_DEV - 1, SQ, CHUNK), jnp.bfloat16),
            pltpu.VMEM((B, N_DEV - 1, SQ, CHUNK), jnp.bfloat16),
            pltpu.VMEM((B, SQ, CHUNK), jnp.bfloat16),
            pltpu.VMEM((B, N_DEV - 1, SQ, CHUNK), jnp.bfloat16),
            pltpu.SemaphoreType.DMA,
            pltpu.SemaphoreType.DMA((B, N_DEV - 1)),
            pltpu.SemaphoreType.DMA((B, N_DEV - 1)),
            pltpu.SemaphoreType.DMA((B, N_DEV - 1)),
            pltpu.SemaphoreType.DMA((B, N_DEV - 1)),
        ],
        compiler_params=pltpu.CompilerParams(collective_id=0),
    )(X2, Wq, K2, V2, Wo)
    return out2.reshape(B, SQ, D_MODEL)
